# baseline (speedup 1.0000x reference)
import jax
import jax.numpy as jnp
from jax.experimental import pallas as pl
from jax.experimental.pallas import tpu as pltpu

_N_ROWS = 128
_N_COLS = 100000
_BLOCK_N = 6272
_NB = -(-_N_COLS // _BLOCK_N)

_R1 = (13, 15, 26, 6)
_R2 = (17, 29, 16, 24)


def _rotl(x, r):
    return (x << jnp.uint32(r)) | (x >> jnp.uint32(32 - r))


def _four_rounds(x0, x1, rots):
    for r in rots:
        x0 = x0 + x1
        x1 = _rotl(x1, r) ^ x0
    return x0, x1


def _sampler_kernel(x_ref, out_ref, m_ref, i_ref):
    b = pl.program_id(0)

    @pl.when(b == 0)
    def _init():
        m_ref[...] = jnp.full((_N_ROWS, 1), -jnp.inf, jnp.float32)
        i_ref[...] = jnp.zeros((_N_ROWS, 1), jnp.int32)

    row = jax.lax.broadcasted_iota(jnp.int32, (_N_ROWS, _BLOCK_N), 0)
    col = jax.lax.broadcasted_iota(jnp.int32, (_N_ROWS, _BLOCK_N), 1) + b * _BLOCK_N
    n = (row * _N_COLS + col).astype(jnp.uint32)

    ks0 = jnp.uint32(0)
    ks1 = jnp.uint32(42)
    ks2 = jnp.uint32(0 ^ 42 ^ 0x1BD11BDA)
    x0 = jnp.zeros_like(n)
    x1 = n + ks1
    x0, x1 = _four_rounds(x0, x1, _R1)
    x0, x1 = x0 + ks1, x1 + (ks2 + jnp.uint32(1))
    x0, x1 = _four_rounds(x0, x1, _R2)
    x0, x1 = x0 + ks2, x1 + (ks0 + jnp.uint32(2))
    x0, x1 = _four_rounds(x0, x1, _R1)
    x0, x1 = x0 + ks0, x1 + (ks1 + jnp.uint32(3))
    x0, x1 = _four_rounds(x0, x1, _R2)
    x0, x1 = x0 + ks1, x1 + (ks2 + jnp.uint32(4))
    x0, x1 = _four_rounds(x0, x1, _R1)
    x0, x1 = x0 + ks2, x1 + (ks0 + jnp.uint32(5))
    bits = x0 ^ x1

    fb = (bits >> jnp.uint32(9)) | jnp.uint32(0x3F800000)
    floats = jax.lax.bitcast_convert_type(fb, jnp.float32) - jnp.float32(1.0)
    tiny = jnp.float32(jnp.finfo(jnp.float32).tiny)
    u = jnp.maximum(tiny, floats * (jnp.float32(1.0) - tiny) + tiny)
    g = -jnp.log(-jnp.log(u))

    phi = jnp.where(col < _N_COLS, x_ref[...] + g, -jnp.inf)

    m = jnp.max(phi, axis=1, keepdims=True)
    idx = jnp.min(
        jnp.where(phi == m, col, jnp.int32(2**30)), axis=1, keepdims=True
    )

    better = m > m_ref[...]
    i_ref[...] = jnp.where(better, idx, i_ref[...])
    m_ref[...] = jnp.where(better, m, m_ref[...])

    @pl.when(b == _NB - 1)
    def _done():
        out_ref[...] = i_ref[...]


def kernel(logits):
    out = pl.pallas_call(
        _sampler_kernel,
        grid=(_NB,),
        in_specs=[
            pl.BlockSpec((_N_ROWS, _BLOCK_N), lambda b: (0, b)),
        ],
        out_specs=pl.BlockSpec((_N_ROWS, 1), lambda b: (0, 0)),
        out_shape=jax.ShapeDtypeStruct((_N_ROWS, 1), jnp.int32),
        scratch_shapes=[
            pltpu.VMEM((_N_ROWS, 1), jnp.float32),
            pltpu.VMEM((_N_ROWS, 1), jnp.int32),
        ],
        compiler_params=pltpu.CompilerParams(
            dimension_semantics=("arbitrary",),
        ),
    )(logits)
    return out.reshape(_N_ROWS)
